# baseline (speedup 1.0000x reference)
import functools

import jax
import jax.numpy as jnp
from jax import lax
from jax.experimental import pallas as pl
from jax.experimental.pallas import tpu as pltpu
from jax.experimental.pallas import tpu_sc as plsc

LENGTH = 4096
BATCH = 16
D = 512

NC = 2
NS = 16
NW = NC * NS
ROWS = BATCH * LENGTH
RPW = ROWS // NW
CH = 64
NBUF = 2
NCHUNK = RPW // CH
NG = NCHUNK // NBUF
A = NBUF // 2


def _sc_shuffle(x_flat, ids):
    mesh = plsc.VectorSubcoreMesh(core_axis_name="c", subcore_axis_name="s")

    scratch = [pltpu.VMEM((RPW,), jnp.int32)]
    scratch += [pltpu.VMEM((CH, D), jnp.float32) for _ in range(NBUF)]
    scratch += [pltpu.SemaphoreType.DMA for _ in range(2 * NBUF)]

    @functools.partial(
        pl.kernel,
        mesh=mesh,
        out_type=jax.ShapeDtypeStruct((ROWS, D), jnp.float32),
        scratch_types=scratch,
    )
    def k(x_hbm, ids_hbm, out_hbm, idx_all, *bufs):
        rows = bufs[:NBUF]
        gsem = bufs[NBUF:2 * NBUF]
        wsem = bufs[2 * NBUF:]
        wid = lax.axis_index("s") * NC + lax.axis_index("c")
        base = wid * RPW
        i0 = lax.rem(base, LENGTH)
        b_off = base - i0

        pltpu.sync_copy(ids_hbm.at[pl.ds(i0, RPW)], idx_all)

        def addoff(j, carry):
            sl = pl.ds(j * 16, 16)
            idx_all[sl] = idx_all[sl] + b_off
            return carry

        lax.fori_loop(0, RPW // 16, addoff, 0)

        def gd(c, b):
            return pltpu.make_async_copy(
                x_hbm.at[idx_all.at[pl.ds(c * CH, CH)]], rows[b], gsem[b])

        def wd(c, b):
            return pltpu.make_async_copy(
                rows[b], out_hbm.at[pl.ds(base + c * CH, CH)], wsem[b])

        def step(c, b, reuse_c=None, ahead_c=None):
            bg = (b + A) % NBUF
            if reuse_c is not None:
                wd(reuse_c, bg).wait()
            if ahead_c is not None:
                gd(ahead_c, bg).start()
            gd(c, b).wait()
            wd(c, b).start()

        for b in range(A):
            gd(b, b).start()

        for b in range(NBUF):
            step(b, b, b - A if b >= A else None, b + A)

        def body(g, carry):
            for b in range(NBUF):
                c = g * NBUF + b
                step(c, b, c - A, c + A)
            return carry

        lax.fori_loop(1, NG - 1, body, 0)

        for b in range(NBUF):
            c = NCHUNK - NBUF + b
            step(c, b, c - A, c + A if b < NBUF - A else None)

        for b in range(NBUF - A, NBUF):
            wd(NCHUNK - NBUF + b, b).wait()

    return k(x_flat, ids)


def kernel(inputs, ids_shuffle):
    x_flat = inputs.reshape(ROWS, D)
    ids = ids_shuffle.astype(jnp.int32)
    out = _sc_shuffle(x_flat, ids)
    return out.reshape(BATCH, LENGTH, D)


# speedup vs baseline: 1.0132x; 1.0132x over previous
import functools

import jax
import jax.numpy as jnp
from jax import lax
from jax.experimental import pallas as pl
from jax.experimental.pallas import tpu as pltpu
from jax.experimental.pallas import tpu_sc as plsc

LENGTH = 4096
BATCH = 16
D = 512

NC = 2
NS = 16
NW = NC * NS
ROWS = BATCH * LENGTH
RPW = ROWS // NW
CH = 32
NCHUNK = RPW // CH
NG = NCHUNK // 2


def _sc_shuffle(x_flat, ids):
    mesh = plsc.VectorSubcoreMesh(core_axis_name="c", subcore_axis_name="s")

    scratch = [
        pltpu.VMEM((RPW,), jnp.int32),
        pltpu.VMEM((CH, D), jnp.float32),
        pltpu.VMEM((CH, D), jnp.float32),
        pltpu.VMEM_SHARED((NS, 2, CH, D), jnp.float32),
        pltpu.SemaphoreType.DMA,
        pltpu.SemaphoreType.DMA,
        pltpu.SemaphoreType.DMA,
        pltpu.SemaphoreType.DMA,
        pltpu.SemaphoreType.DMA,
        pltpu.SemaphoreType.DMA,
    ]

    @functools.partial(
        pl.kernel,
        mesh=mesh,
        out_type=jax.ShapeDtypeStruct((ROWS, D), jnp.float32),
        scratch_types=scratch,
    )
    def k(x_hbm, ids_hbm, out_hbm, idx_all, t0, t1, stage,
          g0, g1, x0, x1, w0, w1):
        tbuf = (t0, t1)
        gsem = (g0, g1)
        xsem = (x0, x1)
        wsem = (w0, w1)
        s = lax.axis_index("s")
        wid = s * NC + lax.axis_index("c")
        base = wid * RPW
        i0 = lax.rem(base, LENGTH)
        b_off = base - i0

        pltpu.sync_copy(ids_hbm.at[pl.ds(i0, RPW)], idx_all)

        def addoff(j, carry):
            sl = pl.ds(j * 16, 16)
            idx_all[sl] = idx_all[sl] + b_off
            return carry

        lax.fori_loop(0, RPW // 16, addoff, 0)

        def gd(c, b):
            return pltpu.make_async_copy(
                x_hbm.at[idx_all.at[pl.ds(c * CH, CH)]], tbuf[b], gsem[b])

        def xd(c, b):
            return pltpu.make_async_copy(tbuf[b], stage.at[s, b], xsem[b])

        def wd(c, b):
            return pltpu.make_async_copy(
                stage.at[s, b], out_hbm.at[pl.ds(base + c * CH, CH)], wsem[b])

        def step(c, b, reuse=True, ahead=True):
            gd(c, b).wait()
            if reuse:
                wd(c - 2, b).wait()
            xd(c, b).start()
            xd(c, b).wait()
            if ahead:
                gd(c + 2, b).start()
            wd(c, b).start()

        gd(0, 0).start()
        gd(1, 1).start()

        for b in range(2):
            step(b, b, reuse=False, ahead=True)

        def body(g, carry):
            for b in range(2):
                step(g * 2 + b, b, reuse=True, ahead=True)
            return carry

        lax.fori_loop(1, NG - 1, body, 0)

        for b in range(2):
            step(NCHUNK - 2 + b, b, reuse=True, ahead=False)

        for b in range(2):
            wd(NCHUNK - 2 + b, b).wait()

    return k(x_flat, ids)


def kernel(inputs, ids_shuffle):
    x_flat = inputs.reshape(ROWS, D)
    ids = ids_shuffle.astype(jnp.int32)
    out = _sc_shuffle(x_flat, ids)
    return out.reshape(BATCH, LENGTH, D)
